# Initial kernel scaffold; baseline (speedup 1.0000x reference)
import jax
import jax.numpy as jnp
from jax.experimental import pallas as pl


def kernel(x, W_enc, b_enc, W_dec, b_dec):
    raise NotImplementedError("write your pallas kernel here")


# speedup vs baseline: 14.6501x; 14.6501x over previous
import functools

import jax
import jax.numpy as jnp
from jax.experimental import pallas as pl


def _fused_body(x_ref, we_ref, wd_ref, be_ref, bd_ref,
                f16_ref, f32_ref, r16_ref, r32_ref, *, k_values):
    e = jnp.dot(x_ref[...], we_ref[...], preferred_element_type=jnp.float32)
    e = jnp.maximum(e + be_ref[...], 0.0)

    k_max = max(k_values)
    thresholds = {}
    ew = e
    for i in range(k_max):
        m = jnp.max(ew, axis=1, keepdims=True)
        if (i + 1) in k_values:
            thresholds[i + 1] = m
        if i + 1 < k_max:
            ew = jnp.where(ew >= m, -1.0, ew)

    f16 = jnp.where(e >= thresholds[k_values[0]], e, 0.0)
    f32_ = jnp.where(e >= thresholds[k_values[1]], e, 0.0)
    f16_ref[...] = f16
    f32_ref[...] = f32_

    wd = wd_ref[...]
    bd = bd_ref[...]
    r16_ref[...] = jnp.dot(f16.astype(jnp.bfloat16), wd,
                           preferred_element_type=jnp.float32) + bd
    r32_ref[...] = jnp.dot(f32_.astype(jnp.bfloat16), wd,
                           preferred_element_type=jnp.float32) + bd


def kernel(x, W_enc, b_enc, W_dec, b_dec):
    B, C, H, W = x.shape
    D = W_enc.shape[0]
    N = B * H * W
    k_values = (16, 32)

    x_flat = jnp.transpose(x, (0, 2, 3, 1)).reshape(N, C)
    x_bf = x_flat.astype(jnp.bfloat16)
    we_t = W_enc.T.astype(jnp.bfloat16)
    wd_t = W_dec.T.astype(jnp.bfloat16)
    be = b_enc.reshape(1, D)
    bd = b_dec.reshape(1, C)

    br = 128 if N % 128 == 0 else N
    grid = (N // br,)

    f16, f32_, r16, r32 = pl.pallas_call(
        functools.partial(_fused_body, k_values=k_values),
        grid=grid,
        in_specs=[
            pl.BlockSpec((br, C), lambda i: (i, 0)),
            pl.BlockSpec((C, D), lambda i: (0, 0)),
            pl.BlockSpec((D, C), lambda i: (0, 0)),
            pl.BlockSpec((1, D), lambda i: (0, 0)),
            pl.BlockSpec((1, C), lambda i: (0, 0)),
        ],
        out_specs=[
            pl.BlockSpec((br, D), lambda i: (i, 0)),
            pl.BlockSpec((br, D), lambda i: (i, 0)),
            pl.BlockSpec((br, C), lambda i: (i, 0)),
            pl.BlockSpec((br, C), lambda i: (i, 0)),
        ],
        out_shape=[
            jax.ShapeDtypeStruct((N, D), jnp.float32),
            jax.ShapeDtypeStruct((N, D), jnp.float32),
            jax.ShapeDtypeStruct((N, C), jnp.float32),
            jax.ShapeDtypeStruct((N, C), jnp.float32),
        ],
    )(x_bf, we_t, wd_t, be, bd)

    recon16 = jnp.transpose(r16.reshape(B, H, W, C), (0, 3, 1, 2))
    recon32 = jnp.transpose(r32.reshape(B, H, W, C), (0, 3, 1, 2))
    return (f16, f32_, recon16, recon32)


# speedup vs baseline: 19.9234x; 1.3600x over previous
import functools

import jax
import jax.numpy as jnp
from jax.experimental import pallas as pl
from jax.experimental.pallas import tpu as pltpu

_SORT8 = [(0, 1), (2, 3), (4, 5), (6, 7),
          (0, 2), (1, 3), (4, 6), (5, 7),
          (1, 2), (5, 6),
          (0, 4), (1, 5), (2, 6), (3, 7),
          (2, 4), (3, 5),
          (1, 2), (3, 4), (5, 6)]

_BITONIC8 = [(0, 4), (1, 5), (2, 6), (3, 7),
             (0, 2), (1, 3), (4, 6), (5, 7),
             (0, 1), (2, 3), (4, 5), (6, 7)]


def _apply_network(planes, network):
    p = list(planes)
    for i, j in network:
        hi = jnp.maximum(p[i], p[j])
        lo = jnp.minimum(p[i], p[j])
        p[i], p[j] = hi, lo
    return p


def _merge_top8(a, b):
    c = [jnp.maximum(a[i], b[7 - i]) for i in range(8)]
    return _apply_network(c, _BITONIC8)


def _extract_thresholds(mat, k_values):
    thresholds = {}
    k_max = max(k_values)
    w = mat
    for i in range(k_max):
        m = jnp.max(w, axis=1, keepdims=True)
        if (i + 1) in k_values:
            thresholds[i + 1] = m
        if i + 1 < k_max:
            w = jnp.where(w >= m, -1.0, w)
    return thresholds


def _fused_body(x_ref, we_ref, wd_ref, be_ref, bd_ref,
                f16_ref, f32_ref, r16_ref, r32_ref,
                t16_s, t32_s, *, k_values):
    ka, kb = k_values
    e = jnp.dot(x_ref[...], we_ref[...], preferred_element_type=jnp.float32)
    e = jnp.maximum(e + be_ref[...], 0.0)
    d = e.shape[1]
    nseg = d // 128

    planes = [e[:, 128 * j:128 * (j + 1)] for j in range(nseg)]
    sorted_chunks = [_apply_network(planes[8 * g:8 * (g + 1)], _SORT8)
                     for g in range(nseg // 8)]
    top8 = sorted_chunks[0]
    for chunk in sorted_chunks[1:]:
        top8 = _merge_top8(top8, chunk)
    reduced = jnp.concatenate(top8, axis=1)

    th = _extract_thresholds(reduced, k_values)
    t16_s[...] = th[ka]
    t32_s[...] = th[kb]

    ca = jnp.sum((e >= th[ka]).astype(jnp.int32), axis=1, keepdims=True)
    cb = jnp.sum((e >= th[kb]).astype(jnp.int32), axis=1, keepdims=True)
    ok_a = (ca == ka) | (th[ka] <= 0.0)
    ok_b = (cb == kb) | (th[kb] <= 0.0)
    bad = jnp.logical_not(jnp.all(ok_a & ok_b))

    @pl.when(bad)
    def _fallback():
        th_full = _extract_thresholds(e, k_values)
        t16_s[...] = th_full[ka]
        t32_s[...] = th_full[kb]

    t16 = t16_s[...]
    t32 = t32_s[...]
    f16 = jnp.where(e >= t16, e, 0.0)
    f32_ = jnp.where(e >= t32, e, 0.0)
    f16_ref[...] = f16
    f32_ref[...] = f32_

    wd = wd_ref[...]
    bd = bd_ref[...]
    r16_ref[...] = jnp.dot(f16.astype(jnp.bfloat16), wd,
                           preferred_element_type=jnp.float32) + bd
    r32_ref[...] = jnp.dot(f32_.astype(jnp.bfloat16), wd,
                           preferred_element_type=jnp.float32) + bd


def kernel(x, W_enc, b_enc, W_dec, b_dec):
    B, C, H, W = x.shape
    D = W_enc.shape[0]
    N = B * H * W
    k_values = (16, 32)

    x_flat = jnp.transpose(x, (0, 2, 3, 1)).reshape(N, C)
    x_bf = x_flat.astype(jnp.bfloat16)
    we_t = W_enc.T.astype(jnp.bfloat16)
    wd_t = W_dec.T.astype(jnp.bfloat16)
    be = b_enc.reshape(1, D)
    bd = b_dec.reshape(1, C)

    br = 128 if N % 128 == 0 else N
    grid = (N // br,)

    f16, f32_, r16, r32 = pl.pallas_call(
        functools.partial(_fused_body, k_values=k_values),
        grid=grid,
        in_specs=[
            pl.BlockSpec((br, C), lambda i: (i, 0)),
            pl.BlockSpec((C, D), lambda i: (0, 0)),
            pl.BlockSpec((D, C), lambda i: (0, 0)),
            pl.BlockSpec((1, D), lambda i: (0, 0)),
            pl.BlockSpec((1, C), lambda i: (0, 0)),
        ],
        out_specs=[
            pl.BlockSpec((br, D), lambda i: (i, 0)),
            pl.BlockSpec((br, D), lambda i: (i, 0)),
            pl.BlockSpec((br, C), lambda i: (i, 0)),
            pl.BlockSpec((br, C), lambda i: (i, 0)),
        ],
        out_shape=[
            jax.ShapeDtypeStruct((N, D), jnp.float32),
            jax.ShapeDtypeStruct((N, D), jnp.float32),
            jax.ShapeDtypeStruct((N, C), jnp.float32),
            jax.ShapeDtypeStruct((N, C), jnp.float32),
        ],
        scratch_shapes=[
            pltpu.VMEM((br, 1), jnp.float32),
            pltpu.VMEM((br, 1), jnp.float32),
        ],
    )(x_bf, we_t, wd_t, be, bd)

    recon16 = jnp.transpose(r16.reshape(B, H, W, C), (0, 3, 1, 2))
    recon32 = jnp.transpose(r32.reshape(B, H, W, C), (0, 3, 1, 2))
    return (f16, f32_, recon16, recon32)


# speedup vs baseline: 20.0394x; 1.0058x over previous
import functools

import jax
import jax.numpy as jnp
from jax.experimental import pallas as pl
from jax.experimental.pallas import tpu as pltpu

_SORT8 = [(0, 1), (2, 3), (4, 5), (6, 7),
          (0, 2), (1, 3), (4, 6), (5, 7),
          (1, 2), (5, 6),
          (0, 4), (1, 5), (2, 6), (3, 7),
          (2, 4), (3, 5),
          (1, 2), (3, 4), (5, 6)]

_BITONIC8 = [(0, 4), (1, 5), (2, 6), (3, 7),
             (0, 2), (1, 3), (4, 6), (5, 7),
             (0, 1), (2, 3), (4, 5), (6, 7)]


def _apply_network(planes, network):
    p = list(planes)
    for i, j in network:
        hi = jnp.maximum(p[i], p[j])
        lo = jnp.minimum(p[i], p[j])
        p[i], p[j] = hi, lo
    return p


def _merge_top8(a, b):
    c = [jnp.maximum(a[i], b[7 - i]) for i in range(8)]
    d = jnp.minimum(a[0], b[7])
    for i in range(1, 8):
        d = jnp.maximum(d, jnp.minimum(a[i], b[7 - i]))
    return _apply_network(c, _BITONIC8), d


def _extract_thresholds(mat, k_values):
    thresholds = {}
    k_max = max(k_values)
    w = mat
    for i in range(k_max):
        m = jnp.max(w, axis=1, keepdims=True)
        if (i + 1) in k_values:
            thresholds[i + 1] = m
        if i + 1 < k_max:
            w = jnp.where(w >= m, -1.0, w)
    return thresholds


def _fused_body(x_ref, we_ref, wd_ref, be_ref, bd_ref,
                f16_ref, f32_ref, r16_ref, r32_ref,
                t16_s, t32_s, *, k_values):
    ka, kb = k_values
    e = jnp.dot(x_ref[...], we_ref[...], preferred_element_type=jnp.float32)
    e = jnp.maximum(e + be_ref[...], 0.0)
    d = e.shape[1]
    nseg = d // 128

    planes = [e[:, 128 * j:128 * (j + 1)] for j in range(nseg)]
    sorted_chunks = [_apply_network(planes[8 * g:8 * (g + 1)], _SORT8)
                     for g in range(nseg // 8)]
    top8 = sorted_chunks[0]
    discard_bound = None
    for chunk in sorted_chunks[1:]:
        top8, dmax = _merge_top8(top8, chunk)
        discard_bound = dmax if discard_bound is None else \
            jnp.maximum(discard_bound, dmax)
    reduced = jnp.concatenate(top8, axis=1)

    th = _extract_thresholds(reduced, k_values)
    t16_s[...] = th[ka]
    t32_s[...] = th[kb]

    if discard_bound is not None:
        db = jnp.max(discard_bound, axis=1, keepdims=True)
        ta, tb = th[ka], th[kb]
        bad_row = ((tb > 0.0) & (db >= tb)) | \
                  ((tb <= 0.0) & (ta > 0.0) & (db >= ta))
        bad = jnp.any(bad_row)

        @pl.when(bad)
        def _fallback():
            th_full = _extract_thresholds(e, k_values)
            t16_s[...] = th_full[ka]
            t32_s[...] = th_full[kb]

    t16 = t16_s[...]
    t32 = t32_s[...]
    f16 = jnp.where(e >= t16, e, 0.0)
    f32_ = jnp.where(e >= t32, e, 0.0)
    f16_ref[...] = f16
    f32_ref[...] = f32_

    wd = wd_ref[...]
    bd = bd_ref[...]
    r16_ref[...] = jnp.dot(f16.astype(jnp.bfloat16), wd,
                           preferred_element_type=jnp.float32) + bd
    r32_ref[...] = jnp.dot(f32_.astype(jnp.bfloat16), wd,
                           preferred_element_type=jnp.float32) + bd


def kernel(x, W_enc, b_enc, W_dec, b_dec):
    B, C, H, W = x.shape
    D = W_enc.shape[0]
    N = B * H * W
    k_values = (16, 32)

    x_flat = jnp.transpose(x, (0, 2, 3, 1)).reshape(N, C)
    x_bf = x_flat.astype(jnp.bfloat16)
    we_t = W_enc.T.astype(jnp.bfloat16)
    wd_t = W_dec.T.astype(jnp.bfloat16)
    be = b_enc.reshape(1, D)
    bd = b_dec.reshape(1, C)

    br = 128 if N % 128 == 0 else N
    grid = (N // br,)

    f16, f32_, r16, r32 = pl.pallas_call(
        functools.partial(_fused_body, k_values=k_values),
        grid=grid,
        in_specs=[
            pl.BlockSpec((br, C), lambda i: (i, 0)),
            pl.BlockSpec((C, D), lambda i: (0, 0)),
            pl.BlockSpec((D, C), lambda i: (0, 0)),
            pl.BlockSpec((1, D), lambda i: (0, 0)),
            pl.BlockSpec((1, C), lambda i: (0, 0)),
        ],
        out_specs=[
            pl.BlockSpec((br, D), lambda i: (i, 0)),
            pl.BlockSpec((br, D), lambda i: (i, 0)),
            pl.BlockSpec((br, C), lambda i: (i, 0)),
            pl.BlockSpec((br, C), lambda i: (i, 0)),
        ],
        out_shape=[
            jax.ShapeDtypeStruct((N, D), jnp.float32),
            jax.ShapeDtypeStruct((N, D), jnp.float32),
            jax.ShapeDtypeStruct((N, C), jnp.float32),
            jax.ShapeDtypeStruct((N, C), jnp.float32),
        ],
        scratch_shapes=[
            pltpu.VMEM((br, 1), jnp.float32),
            pltpu.VMEM((br, 1), jnp.float32),
        ],
    )(x_bf, we_t, wd_t, be, bd)

    recon16 = jnp.transpose(r16.reshape(B, H, W, C), (0, 3, 1, 2))
    recon32 = jnp.transpose(r32.reshape(B, H, W, C), (0, 3, 1, 2))
    return (f16, f32_, recon16, recon32)


# speedup vs baseline: 20.1956x; 1.0078x over previous
import functools

import jax
import jax.numpy as jnp
from jax.experimental import pallas as pl
from jax.experimental.pallas import tpu as pltpu

_SORT8 = [(0, 1), (2, 3), (4, 5), (6, 7),
          (0, 2), (1, 3), (4, 6), (5, 7),
          (1, 2), (5, 6),
          (0, 4), (1, 5), (2, 6), (3, 7),
          (2, 4), (3, 5),
          (1, 2), (3, 4), (5, 6)]

_BITONIC8 = [(0, 4), (1, 5), (2, 6), (3, 7),
             (0, 2), (1, 3), (4, 6), (5, 7),
             (0, 1), (2, 3), (4, 5), (6, 7)]


def _apply_network(planes, network):
    p = list(planes)
    for i, j in network:
        hi = jnp.maximum(p[i], p[j])
        lo = jnp.minimum(p[i], p[j])
        p[i], p[j] = hi, lo
    return p


def _merge_top8(a, b):
    c = [jnp.maximum(a[i], b[7 - i]) for i in range(8)]
    d = jnp.minimum(a[0], b[7])
    for i in range(1, 8):
        d = jnp.maximum(d, jnp.minimum(a[i], b[7 - i]))
    return _apply_network(c, _BITONIC8), d


def _extract_thresholds(mat, k_values):
    thresholds = {}
    k_max = max(k_values)
    w = mat
    for i in range(k_max):
        m = jnp.max(w, axis=1, keepdims=True)
        if (i + 1) in k_values:
            thresholds[i + 1] = m
        if i + 1 < k_max:
            w = jnp.where(w >= m, -1.0, w)
    return thresholds


def _fused_body(x_ref, we_ref, wd_ref, be_ref, bd_ref,
                f16_ref, f32_ref, r16_ref, r32_ref,
                t16_s, t32_s, *, k_values):
    ka, kb = k_values
    e = jnp.dot(x_ref[...], we_ref[...], preferred_element_type=jnp.float32)
    e = jnp.maximum(e + be_ref[...], 0.0)
    br, d = e.shape
    nseg = d // 128

    bad_any = False
    for r in range(0, br, 8):
        er = e[r:r + 8, :]
        planes = [er[:, 128 * j:128 * (j + 1)] for j in range(nseg)]
        sorted_chunks = [_apply_network(planes[8 * g:8 * (g + 1)], _SORT8)
                         for g in range(nseg // 8)]
        top8 = sorted_chunks[0]
        discard_bound = None
        for chunk in sorted_chunks[1:]:
            top8, dmax = _merge_top8(top8, chunk)
            discard_bound = dmax if discard_bound is None else \
                jnp.maximum(discard_bound, dmax)
        reduced = jnp.concatenate(top8, axis=1)

        th = _extract_thresholds(reduced, k_values)
        t16_s[r:r + 8, :] = th[ka]
        t32_s[r:r + 8, :] = th[kb]

        if discard_bound is not None:
            db = jnp.max(discard_bound, axis=1, keepdims=True)
            ta, tb = th[ka], th[kb]
            bad_row = ((tb > 0.0) & (db >= tb)) | \
                      ((tb <= 0.0) & (ta > 0.0) & (db >= ta))
            bad_any = jnp.any(bad_row) | bad_any

    if not isinstance(bad_any, bool):
        @pl.when(bad_any)
        def _fallback():
            th_full = _extract_thresholds(e, k_values)
            t16_s[...] = th_full[ka]
            t32_s[...] = th_full[kb]

    t16 = t16_s[...]
    t32 = t32_s[...]
    f16 = jnp.where(e >= t16, e, 0.0)
    f32_ = jnp.where(e >= t32, e, 0.0)
    f16_ref[...] = f16
    f32_ref[...] = f32_

    wd = wd_ref[...]
    bd = bd_ref[...]
    r16_ref[...] = jnp.dot(f16.astype(jnp.bfloat16), wd,
                           preferred_element_type=jnp.float32) + bd
    r32_ref[...] = jnp.dot(f32_.astype(jnp.bfloat16), wd,
                           preferred_element_type=jnp.float32) + bd


def kernel(x, W_enc, b_enc, W_dec, b_dec):
    B, C, H, W = x.shape
    D = W_enc.shape[0]
    N = B * H * W
    k_values = (16, 32)

    x_flat = jnp.transpose(x, (0, 2, 3, 1)).reshape(N, C)
    x_bf = x_flat.astype(jnp.bfloat16)
    we_t = W_enc.T.astype(jnp.bfloat16)
    wd_t = W_dec.T.astype(jnp.bfloat16)
    be = b_enc.reshape(1, D)
    bd = b_dec.reshape(1, C)

    br = 128 if N % 128 == 0 else N
    grid = (N // br,)

    f16, f32_, r16, r32 = pl.pallas_call(
        functools.partial(_fused_body, k_values=k_values),
        grid=grid,
        in_specs=[
            pl.BlockSpec((br, C), lambda i: (i, 0)),
            pl.BlockSpec((C, D), lambda i: (0, 0)),
            pl.BlockSpec((D, C), lambda i: (0, 0)),
            pl.BlockSpec((1, D), lambda i: (0, 0)),
            pl.BlockSpec((1, C), lambda i: (0, 0)),
        ],
        out_specs=[
            pl.BlockSpec((br, D), lambda i: (i, 0)),
            pl.BlockSpec((br, D), lambda i: (i, 0)),
            pl.BlockSpec((br, C), lambda i: (i, 0)),
            pl.BlockSpec((br, C), lambda i: (i, 0)),
        ],
        out_shape=[
            jax.ShapeDtypeStruct((N, D), jnp.float32),
            jax.ShapeDtypeStruct((N, D), jnp.float32),
            jax.ShapeDtypeStruct((N, C), jnp.float32),
            jax.ShapeDtypeStruct((N, C), jnp.float32),
        ],
        scratch_shapes=[
            pltpu.VMEM((br, 1), jnp.float32),
            pltpu.VMEM((br, 1), jnp.float32),
        ],
    )(x_bf, we_t, wd_t, be, bd)

    recon16 = jnp.transpose(r16.reshape(B, H, W, C), (0, 3, 1, 2))
    recon32 = jnp.transpose(r32.reshape(B, H, W, C), (0, 3, 1, 2))
    return (f16, f32_, recon16, recon32)


# speedup vs baseline: 23.5364x; 1.1654x over previous
import functools

import jax
import jax.numpy as jnp
from jax import lax
from jax.experimental import pallas as pl
from jax.experimental.pallas import tpu as pltpu

_SORT8 = [(0, 1), (2, 3), (4, 5), (6, 7),
          (0, 2), (1, 3), (4, 6), (5, 7),
          (1, 2), (5, 6),
          (0, 4), (1, 5), (2, 6), (3, 7),
          (2, 4), (3, 5),
          (1, 2), (3, 4), (5, 6)]

_BITONIC8 = [(0, 4), (1, 5), (2, 6), (3, 7),
             (0, 2), (1, 3), (4, 6), (5, 7),
             (0, 1), (2, 3), (4, 5), (6, 7)]

_K_VALS = (16, 32)


def _apply_network(planes, network):
    p = list(planes)
    for i, j in network:
        hi = jnp.maximum(p[i], p[j])
        lo = jnp.minimum(p[i], p[j])
        p[i], p[j] = hi, lo
    return p


def _merge_top8(a, b):
    c = [jnp.maximum(a[i], b[7 - i]) for i in range(8)]
    d = jnp.minimum(a[0], b[7])
    for i in range(1, 8):
        d = jnp.maximum(d, jnp.minimum(a[i], b[7 - i]))
    return _apply_network(c, _BITONIC8), d


def _extract_thresholds(mat, k_values):
    thresholds = {}
    k_max = max(k_values)
    w = mat
    for i in range(k_max):
        m = jnp.max(w, axis=1, keepdims=True)
        if (i + 1) in k_values:
            thresholds[i + 1] = m
        if i + 1 < k_max:
            w = jnp.where(w >= m, -1.0, w)
    return thresholds


def _encode(x_ref, we_ref, be_ref):
    e = jnp.dot(x_ref[...], we_ref[...], preferred_element_type=jnp.float32)
    return jnp.maximum(e + be_ref[...], 0.0)


def _mask_decode(e, t16, t32, wd_ref, bd_ref,
                 f16_ref, f32_ref, r16_ref, r32_ref):
    f16 = jnp.where(e >= t16, e, 0.0)
    f32_ = jnp.where(e >= t32, e, 0.0)
    f16_ref[...] = f16
    f32_ref[...] = f32_
    wd = wd_ref[...]
    bd = bd_ref[...]
    r16_ref[...] = jnp.dot(f16.astype(jnp.bfloat16), wd,
                           preferred_element_type=jnp.float32) + bd
    r32_ref[...] = jnp.dot(f32_.astype(jnp.bfloat16), wd,
                           preferred_element_type=jnp.float32) + bd


def _fast_body(x_ref, we_ref, wd_ref, be_ref, bd_ref,
               f16_ref, f32_ref, r16_ref, r32_ref, flag_ref,
               t16_s, t32_s):
    ka, kb = _K_VALS
    e = _encode(x_ref, we_ref, be_ref)
    br, d = e.shape
    nseg = d // 128

    bad_any = None
    for r in range(0, br, 8):
        er = e[r:r + 8, :]
        planes = [er[:, 128 * j:128 * (j + 1)] for j in range(nseg)]
        sorted_chunks = [_apply_network(planes[8 * g:8 * (g + 1)], _SORT8)
                         for g in range(nseg // 8)]
        top8 = sorted_chunks[0]
        discard_bound = None
        for chunk in sorted_chunks[1:]:
            top8, dmax = _merge_top8(top8, chunk)
            discard_bound = dmax if discard_bound is None else \
                jnp.maximum(discard_bound, dmax)
        reduced = jnp.concatenate(top8, axis=1)

        th = _extract_thresholds(reduced, _K_VALS)
        t16_s[r:r + 8, :] = th[ka]
        t32_s[r:r + 8, :] = th[kb]

        if discard_bound is not None:
            db = jnp.max(discard_bound, axis=1, keepdims=True)
            ta, tb = th[ka], th[kb]
            bad_row = ((tb > 0.0) & (db >= tb)) | \
                      ((tb <= 0.0) & (ta > 0.0) & (db >= ta))
            any_r = jnp.max(bad_row.astype(jnp.float32))
            bad_any = any_r if bad_any is None else \
                jnp.maximum(bad_any, any_r)

    if bad_any is None:
        bad_any = jnp.float32(0.0)
    flag_ref[...] = jnp.broadcast_to(bad_any, flag_ref.shape)

    _mask_decode(e, t16_s[...], t32_s[...], wd_ref, bd_ref,
                 f16_ref, f32_ref, r16_ref, r32_ref)


def _exact_body(x_ref, we_ref, wd_ref, be_ref, bd_ref,
                f16_ref, f32_ref, r16_ref, r32_ref):
    ka, kb = _K_VALS
    e = _encode(x_ref, we_ref, be_ref)
    th = _extract_thresholds(e, _K_VALS)
    _mask_decode(e, th[ka], th[kb], wd_ref, bd_ref,
                 f16_ref, f32_ref, r16_ref, r32_ref)


def _make_call(body, br, C, D, N, with_flag):
    grid = (N // br,)
    out_specs = [
        pl.BlockSpec((br, D), lambda i: (i, 0)),
        pl.BlockSpec((br, D), lambda i: (i, 0)),
        pl.BlockSpec((br, C), lambda i: (i, 0)),
        pl.BlockSpec((br, C), lambda i: (i, 0)),
    ]
    out_shape = [
        jax.ShapeDtypeStruct((N, D), jnp.float32),
        jax.ShapeDtypeStruct((N, D), jnp.float32),
        jax.ShapeDtypeStruct((N, C), jnp.float32),
        jax.ShapeDtypeStruct((N, C), jnp.float32),
    ]
    scratch = []
    if with_flag:
        out_specs.append(pl.BlockSpec((1, 1, 128), lambda i: (i, 0, 0)))
        out_shape.append(jax.ShapeDtypeStruct((N // br, 1, 128), jnp.float32))
        scratch = [pltpu.VMEM((br, 1), jnp.float32),
                   pltpu.VMEM((br, 1), jnp.float32)]
    return pl.pallas_call(
        body,
        grid=grid,
        in_specs=[
            pl.BlockSpec((br, C), lambda i: (i, 0)),
            pl.BlockSpec((C, D), lambda i: (0, 0)),
            pl.BlockSpec((D, C), lambda i: (0, 0)),
            pl.BlockSpec((1, D), lambda i: (0, 0)),
            pl.BlockSpec((1, C), lambda i: (0, 0)),
        ],
        out_specs=out_specs,
        out_shape=out_shape,
        scratch_shapes=scratch,
    )


def kernel(x, W_enc, b_enc, W_dec, b_dec):
    B, C, H, W = x.shape
    D = W_enc.shape[0]
    N = B * H * W

    x_flat = jnp.transpose(x, (0, 2, 3, 1)).reshape(N, C)
    x_bf = x_flat.astype(jnp.bfloat16)
    we_t = W_enc.T.astype(jnp.bfloat16)
    wd_t = W_dec.T.astype(jnp.bfloat16)
    be = b_enc.reshape(1, D)
    bd = b_dec.reshape(1, C)

    br = 128 if N % 128 == 0 else N
    args = (x_bf, we_t, wd_t, be, bd)

    f16, f32_, r16, r32, flags = _make_call(
        _fast_body, br, C, D, N, with_flag=True)(*args)

    def _rare(_):
        return tuple(_make_call(_exact_body, br, C, D, N, with_flag=False)(*args))

    f16, f32_, r16, r32 = lax.cond(
        jnp.max(flags) > 0.0, _rare,
        lambda _: (f16, f32_, r16, r32), operand=None)

    recon16 = jnp.transpose(r16.reshape(B, H, W, C), (0, 3, 1, 2))
    recon32 = jnp.transpose(r32.reshape(B, H, W, C), (0, 3, 1, 2))
    return (f16, f32_, recon16, recon32)
